# baseline (speedup 1.0000x reference)
import functools

import jax
import jax.numpy as jnp
from jax import lax
from jax.experimental import pallas as pl
from jax.experimental.pallas import tpu as pltpu
from jax.experimental.pallas import tpu_sc as plsc

B = 16384
D = 64
L = 16
NC = 2
NS = 16
NW = NC * NS
BPW = B // NW
CHUNK = 128
NCHUNK = BPW // CHUNK
FBLK = 4

_mesh = plsc.VectorSubcoreMesh(core_axis_name="c", subcore_axis_name="s")


@functools.partial(
    pl.kernel,
    mesh=_mesh,
    compiler_params=pltpu.CompilerParams(
        needs_layout_passes=False, use_tc_tiling_on_sc=False),
    out_type=jax.ShapeDtypeStruct((B,), jnp.float32),
    scratch_types=[
        pltpu.VMEM((NCHUNK, CHUNK), jnp.int32),
        pltpu.VMEM((NCHUNK, CHUNK), jnp.int32),
        pltpu.VMEM((D, BPW), jnp.float32),
        pltpu.VMEM((D, BPW), jnp.float32),
        pltpu.VMEM((D, L), jnp.float32),
        pltpu.VMEM((L,), jnp.float32),
        pltpu.VMEM((BPW,), jnp.float32),
        pltpu.SemaphoreType.DMA,
    ],
)
def _gmf_sc(uid_hbm, iid_hbm, utabT, itabT, w_hbm, bias_hbm, out_hbm,
            uidx, iidx, ucols, icols, wv, bv, outv, sem):
    wid = lax.axis_index("s") * NC + lax.axis_index("c")
    base = wid * BPW

    pltpu.sync_copy(uid_hbm.at[wid], uidx)
    pltpu.sync_copy(iid_hbm.at[wid], iidx)
    pltpu.sync_copy(w_hbm, wv)
    pltpu.sync_copy(bias_hbm, bv)

    pending = []
    for fb in range(D // FBLK):
        blk = []
        for j in range(FBLK):
            f = fb * FBLK + j
            for c in range(NCHUNK):
                dst = pl.ds(c * CHUNK, CHUNK)
                blk.append(pltpu.async_copy(
                    utabT.at[f].at[uidx.at[c]], ucols.at[f, dst], sem))
                blk.append(pltpu.async_copy(
                    itabT.at[f].at[iidx.at[c]], icols.at[f, dst], sem))
        if pending:
            for cp in pending:
                cp.wait()
        pending = blk
    for cp in pending:
        cp.wait()

    for f in range(D):
        wsplat = wv[f, pl.ds(0, L)]

        def scale(g, carry, f=f, wsplat=wsplat):
            bs = pl.ds(g * L, L)
            ucols[f, bs] = ucols[f, bs] * wsplat
            return carry

        lax.fori_loop(0, BPW // L, scale, 0)

    bias = bv[...]

    def group(g, carry):
        bs = pl.ds(g * L, L)
        acc = bias
        for f in range(D):
            acc = acc + ucols[f, bs] * icols[f, bs]
        outv[bs] = acc
        return carry

    lax.fori_loop(0, BPW // L, group, 0)

    pltpu.sync_copy(outv, out_hbm.at[pl.ds(base, BPW)])


def kernel(user_ids, item_ids, user_table, item_table, W, b):
    uid = user_ids.astype(jnp.int32).reshape(NW, NCHUNK, CHUNK)
    iid = item_ids.astype(jnp.int32).reshape(NW, NCHUNK, CHUNK)
    utabT = user_table.T
    itabT = item_table.T
    wmat = jnp.broadcast_to(W.reshape(D, 1).astype(jnp.float32), (D, L))
    bias = jnp.full((L,), b[0], dtype=jnp.float32)
    return _gmf_sc(uid, iid, utabT, itabT, wmat, bias)


# speedup vs baseline: 12.1492x; 12.1492x over previous
import functools

import jax
import jax.numpy as jnp
from jax import lax
from jax.experimental import pallas as pl
from jax.experimental.pallas import tpu as pltpu
from jax.experimental.pallas import tpu_sc as plsc

B = 16384
D = 64
L = 16
NC = 2
NS = 16
NW = NC * NS
BPW = B // NW
CHUNK = 128
NCHUNK = BPW // CHUNK
FBLK = 4

NU = 100000
NI = 1000000
PU = 100096
PI = 1000064
FCT_U = NU // 128
FCT_I = NI // 128
TAIL_U = NU - FCT_U * 128
TAIL_I = NI - FCT_I * 128
SH_U = 25
SH_I = 245

_mesh = plsc.VectorSubcoreMesh(core_axis_name="c", subcore_axis_name="s")


@functools.partial(
    pl.kernel,
    mesh=_mesh,
    compiler_params=pltpu.CompilerParams(
        needs_layout_passes=False, use_tc_tiling_on_sc=True),
    out_type=(jax.ShapeDtypeStruct((D * PU,), jnp.float32),
              jax.ShapeDtypeStruct((D * PI,), jnp.float32)),
    scratch_types=[
        pltpu.VMEM((2, SH_I * 128), jnp.float32),
        pltpu.SemaphoreType.DMA,
        pltpu.SemaphoreType.DMA,
    ],
)
def _detile_sc(utabT, itabT, tail_u, tail_i, ulin, ilin, vbuf, rsem, wsem):
    wid = lax.axis_index("s") * NC + lax.axis_index("c")
    su = jnp.minimum(wid * SH_U, FCT_U - SH_U) * 128
    si = jnp.minimum(wid * SH_I, FCT_I - SH_I) * 128

    wr = [None, None]
    for f in range(D):
        p = f % 2
        if wr[p] is not None:
            wr[p].wait()
        src = vbuf.at[p, pl.ds(0, SH_I * 128)]
        pltpu.async_copy(
            itabT.at[f, pl.ds(si, SH_I * 128)], src, rsem).wait()
        wr[p] = pltpu.async_copy(
            src, ilin.at[pl.ds(f * PI + si, SH_I * 128)], wsem)
    for cp in wr:
        cp.wait()

    wr = [None, None]
    for f in range(D):
        p = f % 2
        if wr[p] is not None:
            wr[p].wait()
        src = vbuf.at[p, pl.ds(0, SH_U * 128)]
        pltpu.async_copy(
            utabT.at[f, pl.ds(su, SH_U * 128)], src, rsem).wait()
        wr[p] = pltpu.async_copy(
            src, ulin.at[pl.ds(f * PU + su, SH_U * 128)], wsem)
    for cp in wr:
        cp.wait()

    @pl.when(wid == 0)
    def _tails():
        tu = vbuf.at[0, pl.ds(0, D * TAIL_U)]
        pltpu.sync_copy(tail_u, tu)
        ti = vbuf.at[1, pl.ds(0, D * TAIL_I)]
        pltpu.sync_copy(tail_i, ti)
        tcps = []
        for f in range(D):
            tcps.append(pltpu.async_copy(
                tu.at[pl.ds(f * TAIL_U, TAIL_U)],
                ulin.at[pl.ds(f * PU + FCT_U * 128, TAIL_U)], wsem))
            tcps.append(pltpu.async_copy(
                ti.at[pl.ds(f * TAIL_I, TAIL_I)],
                ilin.at[pl.ds(f * PI + FCT_I * 128, TAIL_I)], wsem))
        for cp in tcps:
            cp.wait()


@functools.partial(
    pl.kernel,
    mesh=_mesh,
    compiler_params=pltpu.CompilerParams(
        needs_layout_passes=False, use_tc_tiling_on_sc=False),
    out_type=jax.ShapeDtypeStruct((B,), jnp.float32),
    scratch_types=[
        pltpu.VMEM((NCHUNK, CHUNK), jnp.int32),
        pltpu.VMEM((NCHUNK, CHUNK), jnp.int32),
        pltpu.VMEM((D, BPW), jnp.float32),
        pltpu.VMEM((D, BPW), jnp.float32),
        pltpu.VMEM((D, L), jnp.float32),
        pltpu.VMEM((L,), jnp.float32),
        pltpu.VMEM((BPW,), jnp.float32),
        pltpu.SemaphoreType.DMA,
    ],
)
def _gmf_sc(uid_hbm, iid_hbm, ulin, ilin, w_hbm, bias_hbm, out_hbm,
            uidx, iidx, ucols, icols, wv, bv, outv, sem):
    wid = lax.axis_index("s") * NC + lax.axis_index("c")
    base = wid * BPW

    pltpu.sync_copy(uid_hbm.at[wid], uidx)
    pltpu.sync_copy(iid_hbm.at[wid], iidx)
    pltpu.sync_copy(w_hbm, wv)
    pltpu.sync_copy(bias_hbm, bv)

    pending = []
    for fb in range(D // FBLK):
        blk = []
        for j in range(FBLK):
            f = fb * FBLK + j
            for c in range(NCHUNK):
                dst = pl.ds(c * CHUNK, CHUNK)
                blk.append(pltpu.async_copy(
                    ulin.at[pl.ds(f * PU, PU)].at[uidx.at[c]],
                    ucols.at[f, dst], sem))
                blk.append(pltpu.async_copy(
                    ilin.at[pl.ds(f * PI, PI)].at[iidx.at[c]],
                    icols.at[f, dst], sem))
        if pending:
            for cp in pending:
                cp.wait()
        pending = blk
    for cp in pending:
        cp.wait()

    for f in range(D):
        wsplat = wv[f, pl.ds(0, L)]

        def scale(g, carry, f=f, wsplat=wsplat):
            bs = pl.ds(g * L, L)
            ucols[f, bs] = ucols[f, bs] * wsplat
            return carry

        lax.fori_loop(0, BPW // L, scale, 0)

    bias = bv[...]

    def group(g, carry):
        bs = pl.ds(g * L, L)
        acc = bias
        for f in range(D):
            acc = acc + ucols[f, bs] * icols[f, bs]
        outv[bs] = acc
        return carry

    lax.fori_loop(0, BPW // L, group, 0)

    pltpu.sync_copy(outv, out_hbm.at[pl.ds(base, BPW)])


def kernel(user_ids, item_ids, user_table, item_table, W, b):
    uid = user_ids.astype(jnp.int32).reshape(NW, NCHUNK, CHUNK)
    iid = item_ids.astype(jnp.int32).reshape(NW, NCHUNK, CHUNK)
    tail_u = user_table[FCT_U * 128:].T.reshape(-1)
    tail_i = item_table[FCT_I * 128:].T.reshape(-1)
    ulin, ilin = _detile_sc(user_table.T, item_table.T, tail_u, tail_i)
    wmat = jnp.broadcast_to(W.reshape(D, 1).astype(jnp.float32), (D, L))
    bias = jnp.full((L,), b[0], dtype=jnp.float32)
    return _gmf_sc(uid, iid, ulin, ilin, wmat, bias)


# speedup vs baseline: 14.8489x; 1.2222x over previous
import functools

import jax
import jax.numpy as jnp
from jax import lax
from jax.experimental import pallas as pl
from jax.experimental.pallas import tpu as pltpu
from jax.experimental.pallas import tpu_sc as plsc

B = 16384
D = 64
L = 16
NC = 2
NS = 16
NW = NC * NS
BPW = B // NW
CHUNK = 128
NCHUNK = BPW // CHUNK
FBLK = 4

NU = 100000
NI = 1000000
PU = 100096
PI = 1000064
FCT_U = NU // 128
FCT_I = NI // 128
TAIL_U = NU - FCT_U * 128
TAIL_I = NI - FCT_I * 128
SH_U = 25
SH_I = 245

_mesh = plsc.VectorSubcoreMesh(core_axis_name="c", subcore_axis_name="s")


@functools.partial(
    pl.kernel,
    mesh=_mesh,
    compiler_params=pltpu.CompilerParams(
        needs_layout_passes=False, use_tc_tiling_on_sc=True),
    out_type=(jax.ShapeDtypeStruct((D * PU,), jnp.float32),
              jax.ShapeDtypeStruct((D * PI,), jnp.float32)),
    scratch_types=[
        pltpu.VMEM((2, SH_I * 128), jnp.float32),
        pltpu.SemaphoreType.DMA,
        pltpu.SemaphoreType.DMA,
    ],
)
def _detile_sc(utabT, itabT, tail_u, tail_i, ulin, ilin, vbuf, rsem, wsem):
    wid = lax.axis_index("s") * NC + lax.axis_index("c")
    su = jnp.minimum(wid * SH_U, FCT_U - SH_U) * 128
    si = jnp.minimum(wid * SH_I, FCT_I - SH_I) * 128

    def relay(tab, lin, start, sz, pitch):
        rd = [None, None]
        wr = [None, None]
        bufs = [vbuf.at[p, pl.ds(0, sz)] for p in range(2)]
        rd[0] = pltpu.async_copy(tab.at[0, pl.ds(start, sz)], bufs[0], rsem)
        for f in range(D):
            p = f % 2
            q = (f + 1) % 2
            if f + 1 < D:
                if wr[q] is not None:
                    wr[q].wait()
                rd[q] = pltpu.async_copy(
                    tab.at[f + 1, pl.ds(start, sz)], bufs[q], rsem)
            rd[p].wait()
            wr[p] = pltpu.async_copy(
                bufs[p], lin.at[pl.ds(f * pitch + start, sz)], wsem)
        for cp in wr:
            cp.wait()

    relay(itabT, ilin, si, SH_I * 128, PI)
    relay(utabT, ulin, su, SH_U * 128, PU)

    @pl.when(wid == 0)
    def _tails():
        tu = vbuf.at[0, pl.ds(0, D * TAIL_U)]
        pltpu.sync_copy(tail_u, tu)
        ti = vbuf.at[1, pl.ds(0, D * TAIL_I)]
        pltpu.sync_copy(tail_i, ti)
        tcps = []
        for f in range(D):
            tcps.append(pltpu.async_copy(
                tu.at[pl.ds(f * TAIL_U, TAIL_U)],
                ulin.at[pl.ds(f * PU + FCT_U * 128, TAIL_U)], wsem))
            tcps.append(pltpu.async_copy(
                ti.at[pl.ds(f * TAIL_I, TAIL_I)],
                ilin.at[pl.ds(f * PI + FCT_I * 128, TAIL_I)], wsem))
        for cp in tcps:
            cp.wait()


@functools.partial(
    pl.kernel,
    mesh=_mesh,
    compiler_params=pltpu.CompilerParams(
        needs_layout_passes=False, use_tc_tiling_on_sc=False),
    out_type=jax.ShapeDtypeStruct((B,), jnp.float32),
    scratch_types=[
        pltpu.VMEM((NCHUNK, CHUNK), jnp.int32),
        pltpu.VMEM((NCHUNK, CHUNK), jnp.int32),
        pltpu.VMEM((D, BPW), jnp.float32),
        pltpu.VMEM((D, BPW), jnp.float32),
        pltpu.VMEM((D, L), jnp.float32),
        pltpu.VMEM((L,), jnp.float32),
        pltpu.VMEM((BPW,), jnp.float32),
        pltpu.SemaphoreType.DMA,
    ],
)
def _gmf_sc(uid_hbm, iid_hbm, ulin, ilin, w_hbm, bias_hbm, out_hbm,
            uidx, iidx, ucols, icols, wv, bv, outv, sem):
    wid = lax.axis_index("s") * NC + lax.axis_index("c")
    base = wid * BPW

    pltpu.sync_copy(uid_hbm.at[wid], uidx)
    pltpu.sync_copy(iid_hbm.at[wid], iidx)
    pltpu.sync_copy(w_hbm, wv)
    pltpu.sync_copy(bias_hbm, bv)

    pending = []
    for fb in range(D // FBLK):
        blk = []
        for j in range(FBLK):
            f = fb * FBLK + j
            for c in range(NCHUNK):
                dst = pl.ds(c * CHUNK, CHUNK)
                blk.append(pltpu.async_copy(
                    ulin.at[pl.ds(f * PU, PU)].at[uidx.at[c]],
                    ucols.at[f, dst], sem))
                blk.append(pltpu.async_copy(
                    ilin.at[pl.ds(f * PI, PI)].at[iidx.at[c]],
                    icols.at[f, dst], sem))
        if pending:
            for cp in pending:
                cp.wait()
        pending = blk
    for cp in pending:
        cp.wait()

    for f in range(D):
        wsplat = wv[f, pl.ds(0, L)]

        def scale(g, carry, f=f, wsplat=wsplat):
            bs = pl.ds(g * L, L)
            ucols[f, bs] = ucols[f, bs] * wsplat
            return carry

        lax.fori_loop(0, BPW // L, scale, 0)

    bias = bv[...]

    def group(g, carry):
        bs = pl.ds(g * L, L)
        acc = bias
        for f in range(D):
            acc = acc + ucols[f, bs] * icols[f, bs]
        outv[bs] = acc
        return carry

    lax.fori_loop(0, BPW // L, group, 0)

    pltpu.sync_copy(outv, out_hbm.at[pl.ds(base, BPW)])


def kernel(user_ids, item_ids, user_table, item_table, W, b):
    uid = user_ids.astype(jnp.int32).reshape(NW, NCHUNK, CHUNK)
    iid = item_ids.astype(jnp.int32).reshape(NW, NCHUNK, CHUNK)
    tail_u = user_table[FCT_U * 128:].T.reshape(-1)
    tail_i = item_table[FCT_I * 128:].T.reshape(-1)
    ulin, ilin = _detile_sc(user_table.T, item_table.T, tail_u, tail_i)
    wmat = jnp.broadcast_to(W.reshape(D, 1).astype(jnp.float32), (D, L))
    bias = jnp.full((L,), b[0], dtype=jnp.float32)
    return _gmf_sc(uid, iid, ulin, ilin, wmat, bias)


# speedup vs baseline: 15.0991x; 1.0169x over previous
import functools

import jax
import jax.numpy as jnp
from jax import lax
from jax.experimental import pallas as pl
from jax.experimental.pallas import tpu as pltpu
from jax.experimental.pallas import tpu_sc as plsc

B = 16384
D = 64
L = 16
NC = 2
NS = 16
NW = NC * NS
BPW = B // NW
CHUNK = 128
NCHUNK = BPW // CHUNK
FBLK = 8

NU = 100000
NI = 1000000
PU = 100096
PI = 1000064
FCT_U = NU // 128
FCT_I = NI // 128
TAIL_U = NU - FCT_U * 128
TAIL_I = NI - FCT_I * 128
SH_U = 25
SH_I = 245

_mesh = plsc.VectorSubcoreMesh(core_axis_name="c", subcore_axis_name="s")


@functools.partial(
    pl.kernel,
    mesh=_mesh,
    compiler_params=pltpu.CompilerParams(
        needs_layout_passes=False, use_tc_tiling_on_sc=True),
    out_type=(jax.ShapeDtypeStruct((D * PU,), jnp.float32),
              jax.ShapeDtypeStruct((D * PI,), jnp.float32)),
    scratch_types=[
        pltpu.VMEM((2, SH_I * 128), jnp.float32),
        pltpu.SemaphoreType.DMA,
        pltpu.SemaphoreType.DMA,
    ],
)
def _detile_sc(utabT, itabT, tail_u, tail_i, ulin, ilin, vbuf, rsem, wsem):
    wid = lax.axis_index("s") * NC + lax.axis_index("c")
    su = jnp.minimum(wid * SH_U, FCT_U - SH_U) * 128
    si = jnp.minimum(wid * SH_I, FCT_I - SH_I) * 128

    def relay(tab, lin, start, sz, pitch):
        rd = [None, None]
        wr = [None, None]
        bufs = [vbuf.at[p, pl.ds(0, sz)] for p in range(2)]
        rd[0] = pltpu.async_copy(tab.at[0, pl.ds(start, sz)], bufs[0], rsem)
        for f in range(D):
            p = f % 2
            q = (f + 1) % 2
            if f + 1 < D:
                if wr[q] is not None:
                    wr[q].wait()
                rd[q] = pltpu.async_copy(
                    tab.at[f + 1, pl.ds(start, sz)], bufs[q], rsem)
            rd[p].wait()
            wr[p] = pltpu.async_copy(
                bufs[p], lin.at[pl.ds(f * pitch + start, sz)], wsem)
        for cp in wr:
            cp.wait()

    relay(itabT, ilin, si, SH_I * 128, PI)
    relay(utabT, ulin, su, SH_U * 128, PU)

    @pl.when(wid == 0)
    def _tails():
        tu = vbuf.at[0, pl.ds(0, D * TAIL_U)]
        pltpu.sync_copy(tail_u, tu)
        ti = vbuf.at[1, pl.ds(0, D * TAIL_I)]
        pltpu.sync_copy(tail_i, ti)
        tcps = []
        for f in range(D):
            tcps.append(pltpu.async_copy(
                tu.at[pl.ds(f * TAIL_U, TAIL_U)],
                ulin.at[pl.ds(f * PU + FCT_U * 128, TAIL_U)], wsem))
            tcps.append(pltpu.async_copy(
                ti.at[pl.ds(f * TAIL_I, TAIL_I)],
                ilin.at[pl.ds(f * PI + FCT_I * 128, TAIL_I)], wsem))
        for cp in tcps:
            cp.wait()


@functools.partial(
    pl.kernel,
    mesh=_mesh,
    compiler_params=pltpu.CompilerParams(
        needs_layout_passes=False, use_tc_tiling_on_sc=False),
    out_type=jax.ShapeDtypeStruct((B,), jnp.float32),
    scratch_types=[
        pltpu.VMEM((NCHUNK, CHUNK), jnp.int32),
        pltpu.VMEM((NCHUNK, CHUNK), jnp.int32),
        pltpu.VMEM((D, BPW), jnp.float32),
        pltpu.VMEM((D, BPW), jnp.float32),
        pltpu.VMEM((D, L), jnp.float32),
        pltpu.VMEM((L,), jnp.float32),
        pltpu.VMEM((BPW,), jnp.float32),
        pltpu.SemaphoreType.DMA,
    ],
)
def _gmf_sc(uid_hbm, iid_hbm, ulin, ilin, w_hbm, bias_hbm, out_hbm,
            uidx, iidx, ucols, icols, wv, bv, outv, sem):
    wid = lax.axis_index("s") * NC + lax.axis_index("c")
    base = wid * BPW

    pltpu.sync_copy(uid_hbm.at[wid], uidx)
    pltpu.sync_copy(iid_hbm.at[wid], iidx)
    pltpu.sync_copy(w_hbm, wv)
    pltpu.sync_copy(bias_hbm, bv)

    pending = []
    for fb in range(D // FBLK):
        blk = []
        for j in range(FBLK):
            f = fb * FBLK + j
            for c in range(NCHUNK):
                dst = pl.ds(c * CHUNK, CHUNK)
                blk.append(pltpu.async_copy(
                    ulin.at[pl.ds(f * PU, PU)].at[uidx.at[c]],
                    ucols.at[f, dst], sem))
                blk.append(pltpu.async_copy(
                    ilin.at[pl.ds(f * PI, PI)].at[iidx.at[c]],
                    icols.at[f, dst], sem))
        if pending:
            for cp in pending:
                cp.wait()
        pending = blk
    for cp in pending:
        cp.wait()

    for f in range(D):
        wsplat = wv[f, pl.ds(0, L)]

        def scale(g, carry, f=f, wsplat=wsplat):
            bs = pl.ds(g * L, L)
            ucols[f, bs] = ucols[f, bs] * wsplat
            return carry

        lax.fori_loop(0, BPW // L, scale, 0)

    bias = bv[...]

    def group(g, carry):
        bs = pl.ds(g * L, L)
        acc = bias
        for f in range(D):
            acc = acc + ucols[f, bs] * icols[f, bs]
        outv[bs] = acc
        return carry

    lax.fori_loop(0, BPW // L, group, 0)

    pltpu.sync_copy(outv, out_hbm.at[pl.ds(base, BPW)])


def kernel(user_ids, item_ids, user_table, item_table, W, b):
    uid = user_ids.astype(jnp.int32).reshape(NW, NCHUNK, CHUNK)
    iid = item_ids.astype(jnp.int32).reshape(NW, NCHUNK, CHUNK)
    tail_u = user_table[FCT_U * 128:].T.reshape(-1)
    tail_i = item_table[FCT_I * 128:].T.reshape(-1)
    ulin, ilin = _detile_sc(user_table.T, item_table.T, tail_u, tail_i)
    wmat = jnp.broadcast_to(W.reshape(D, 1).astype(jnp.float32), (D, L))
    bias = jnp.full((L,), b[0], dtype=jnp.float32)
    return _gmf_sc(uid, iid, ulin, ilin, wmat, bias)


# speedup vs baseline: 15.6939x; 1.0394x over previous
import functools

import jax
import jax.numpy as jnp
from jax import lax
from jax.experimental import pallas as pl
from jax.experimental.pallas import tpu as pltpu
from jax.experimental.pallas import tpu_sc as plsc

B = 16384
D = 64
L = 16
NC = 2
NS = 16
NW = NC * NS
BPW = B // NW
CHUNK = 128
NCHUNK = BPW // CHUNK
FBLK = 8

NU = 100000
NI = 1000000
PU = 100096
PI = 1000064
FCT_U = NU // 128
FCT_I = NI // 128
TAIL_U = NU - FCT_U * 128
TAIL_I = NI - FCT_I * 128
SH_U = 25
SH_I = 245

_mesh = plsc.VectorSubcoreMesh(core_axis_name="c", subcore_axis_name="s")


@functools.partial(
    pl.kernel,
    mesh=_mesh,
    compiler_params=pltpu.CompilerParams(
        needs_layout_passes=False, use_tc_tiling_on_sc=True),
    out_type=(jax.ShapeDtypeStruct((D * PU,), jnp.float32),
              jax.ShapeDtypeStruct((D * PI,), jnp.float32)),
    scratch_types=[
        pltpu.VMEM((4, SH_I * 128), jnp.float32),
        pltpu.SemaphoreType.DMA,
        pltpu.SemaphoreType.DMA,
    ],
)
def _detile_sc(utabT, itabT, tail_u, tail_i, ulin, ilin, vbuf, rsem, wsem):
    wid = lax.axis_index("s") * NC + lax.axis_index("c")
    su = jnp.minimum(wid * SH_U, FCT_U - SH_U) * 128
    si = jnp.minimum(wid * SH_I, FCT_I - SH_I) * 128

    def relay(tab, lin, start, sz, pitch, R=4):
        rd = [None] * R
        wr = [None] * R
        bufs = [vbuf.at[p, pl.ds(0, sz)] for p in range(R)]
        for f in range(R - 1):
            rd[f] = pltpu.async_copy(
                tab.at[f, pl.ds(start, sz)], bufs[f], rsem)
        for f in range(D):
            p = f % R
            nf = f + R - 1
            if nf < D:
                q = nf % R
                if wr[q] is not None:
                    wr[q].wait()
                rd[q] = pltpu.async_copy(
                    tab.at[nf, pl.ds(start, sz)], bufs[q], rsem)
            rd[p].wait()
            wr[p] = pltpu.async_copy(
                bufs[p], lin.at[pl.ds(f * pitch + start, sz)], wsem)
        for cp in wr:
            if cp is not None:
                cp.wait()

    relay(itabT, ilin, si, SH_I * 128, PI)
    relay(utabT, ulin, su, SH_U * 128, PU)

    @pl.when(wid == 0)
    def _tails():
        tu = vbuf.at[0, pl.ds(0, D * TAIL_U)]
        pltpu.sync_copy(tail_u, tu)
        ti = vbuf.at[1, pl.ds(0, D * TAIL_I)]
        pltpu.sync_copy(tail_i, ti)
        tcps = []
        for f in range(D):
            tcps.append(pltpu.async_copy(
                tu.at[pl.ds(f * TAIL_U, TAIL_U)],
                ulin.at[pl.ds(f * PU + FCT_U * 128, TAIL_U)], wsem))
            tcps.append(pltpu.async_copy(
                ti.at[pl.ds(f * TAIL_I, TAIL_I)],
                ilin.at[pl.ds(f * PI + FCT_I * 128, TAIL_I)], wsem))
        for cp in tcps:
            cp.wait()


@functools.partial(
    pl.kernel,
    mesh=_mesh,
    compiler_params=pltpu.CompilerParams(
        needs_layout_passes=False, use_tc_tiling_on_sc=False),
    out_type=jax.ShapeDtypeStruct((B,), jnp.float32),
    scratch_types=[
        pltpu.VMEM((NCHUNK, CHUNK), jnp.int32),
        pltpu.VMEM((NCHUNK, CHUNK), jnp.int32),
        pltpu.VMEM((D, BPW), jnp.float32),
        pltpu.VMEM((D, BPW), jnp.float32),
        pltpu.VMEM((D, L), jnp.float32),
        pltpu.VMEM((L,), jnp.float32),
        pltpu.VMEM((BPW,), jnp.float32),
        pltpu.SemaphoreType.DMA,
    ],
)
def _gmf_sc(uid_hbm, iid_hbm, ulin, ilin, w_hbm, bias_hbm, out_hbm,
            uidx, iidx, ucols, icols, wv, bv, outv, sem):
    wid = lax.axis_index("s") * NC + lax.axis_index("c")
    base = wid * BPW

    pltpu.sync_copy(uid_hbm.at[wid], uidx)
    pltpu.sync_copy(iid_hbm.at[wid], iidx)
    pltpu.sync_copy(w_hbm, wv)
    pltpu.sync_copy(bias_hbm, bv)

    pending = []
    for fb in range(D // FBLK):
        blk = []
        for j in range(FBLK):
            f = fb * FBLK + j
            for c in range(NCHUNK):
                dst = pl.ds(c * CHUNK, CHUNK)
                blk.append(pltpu.async_copy(
                    ulin.at[pl.ds(f * PU, PU)].at[uidx.at[c]],
                    ucols.at[f, dst], sem))
                blk.append(pltpu.async_copy(
                    ilin.at[pl.ds(f * PI, PI)].at[iidx.at[c]],
                    icols.at[f, dst], sem))
        if pending:
            for cp in pending:
                cp.wait()
        pending = blk
    for cp in pending:
        cp.wait()

    for f in range(D):
        wsplat = wv[f, pl.ds(0, L)]

        def scale(g, carry, f=f, wsplat=wsplat):
            bs = pl.ds(g * L, L)
            ucols[f, bs] = ucols[f, bs] * wsplat
            return carry

        lax.fori_loop(0, BPW // L, scale, 0)

    bias = bv[...]

    def group(g, carry):
        bs = pl.ds(g * L, L)
        acc = bias
        for f in range(D):
            acc = acc + ucols[f, bs] * icols[f, bs]
        outv[bs] = acc
        return carry

    lax.fori_loop(0, BPW // L, group, 0)

    pltpu.sync_copy(outv, out_hbm.at[pl.ds(base, BPW)])


def kernel(user_ids, item_ids, user_table, item_table, W, b):
    uid = user_ids.astype(jnp.int32).reshape(NW, NCHUNK, CHUNK)
    iid = item_ids.astype(jnp.int32).reshape(NW, NCHUNK, CHUNK)
    tail_u = user_table[FCT_U * 128:].T.reshape(-1)
    tail_i = item_table[FCT_I * 128:].T.reshape(-1)
    ulin, ilin = _detile_sc(user_table.T, item_table.T, tail_u, tail_i)
    wmat = jnp.broadcast_to(W.reshape(D, 1).astype(jnp.float32), (D, L))
    bias = jnp.full((L,), b[0], dtype=jnp.float32)
    return _gmf_sc(uid, iid, ulin, ilin, wmat, bias)
